# Initial kernel scaffold; baseline (speedup 1.0000x reference)
import jax
import jax.numpy as jnp
from jax.experimental import pallas as pl


def kernel(row_ptr, col_idx, node_feat):
    raise NotImplementedError("write your pallas kernel here")


# speedup vs baseline: 36.5195x; 36.5195x over previous
import functools

import jax
import jax.numpy as jnp
from jax import lax
from jax.experimental import pallas as pl
from jax.experimental.pallas import tpu as pltpu
from jax.experimental.pallas import tpu_sc as plsc

N_NODES = 10000
N_EDGES = 160000
D = 256
NW = 32
NPW = 320
B = 64
RP_PAD = NW * NPW + 16
COL_PAD = N_EDGES + 128
NEG_INF = float("-inf")


def _body(rp_hbm, col_hbm, feat_hbm, out_hbm, rp_v, idx_v, seg_v, rows_v,
          out_v, sem):
    wid = lax.axis_index("s") * 2 + lax.axis_index("c")
    nbase = pl.multiple_of(wid * NPW, 8)
    pltpu.sync_copy(rp_hbm.at[pl.ds(nbase, NPW + 16)], rp_v)
    e_lo = rp_v[pl.ds(0, 16)][0]
    e_hi = rp_v[pl.ds(NPW, 16)][0]
    base8 = lax.bitwise_and(e_lo, -8)
    nblk = lax.div(e_hi - base8 + (B - 1), B)

    ninf = jnp.full((16,), NEG_INF, jnp.float32)

    def init_row(n, _):
        for k in range(D // 16):
            out_v[n, pl.ds(16 * k, 16)] = ninf
        return 0

    lax.fori_loop(0, NPW + 1, init_row, 0)

    def blk_body(b, carry):
        estart = pl.multiple_of(base8 + b * B, 8)
        pltpu.sync_copy(col_hbm.at[pl.ds(estart, B)], idx_v)
        pltpu.async_copy(feat_hbm.at[idx_v], rows_v, sem).wait()

        for g in range(B // 16):
            evec = estart + g * 16 + lax.iota(jnp.int32, 16)
            pos = jnp.zeros((16,), jnp.int32)
            for step in (256, 128, 64, 32, 16, 8, 4, 2, 1):
                cand = pos + step
                candc = jnp.minimum(cand, NPW)
                vals = plsc.load_gather(rp_v, [candc])
                take = (cand <= NPW) & (vals <= evec)
                pos = jnp.where(take, cand, pos)
            valid = (evec >= e_lo) & (evec < e_hi)
            seg = jnp.where(valid, pos, NPW)
            seg_v[pl.ds(g * 16, 16)] = seg

        def edge_body(i, ec):
            cur = ec[0]
            accs = ec[1:]
            seg = seg_v[pl.ds(i, 16)][0]
            flush = seg != cur

            @pl.when(flush)
            def _():
                for k in range(D // 16):
                    out_v[cur, pl.ds(16 * k, 16)] = accs[k]

            new = []
            for k in range(D // 16):
                row = rows_v[i, pl.ds(16 * k, 16)]
                new.append(jnp.where(flush, row, jnp.maximum(accs[k], row)))
            return (seg,) + tuple(new)

        return lax.fori_loop(0, B, edge_body, carry)

    carry0 = (jnp.int32(NPW),) + tuple(ninf for _ in range(D // 16))
    carry = lax.fori_loop(0, nblk, blk_body, carry0)

    cur = carry[0]
    for k in range(D // 16):
        out_v[cur, pl.ds(16 * k, 16)] = carry[1 + k]

    pltpu.sync_copy(out_v.at[pl.ds(0, NPW)], out_hbm.at[pl.ds(nbase, NPW)])


@jax.jit
def kernel(row_ptr, col_idx, node_feat):
    rp_pad = jnp.concatenate(
        [row_ptr,
         jnp.broadcast_to(row_ptr[-1], (RP_PAD - (N_NODES + 1),))])
    col_pad = jnp.concatenate(
        [col_idx, jnp.zeros((COL_PAD - N_EDGES,), jnp.int32)])

    mesh = plsc.VectorSubcoreMesh(core_axis_name="c", subcore_axis_name="s")
    out = pl.kernel(
        _body,
        out_type=jax.ShapeDtypeStruct((NW * NPW, D), jnp.float32),
        mesh=mesh,
        compiler_params=pltpu.CompilerParams(needs_layout_passes=False),
        scratch_types=[
            pltpu.VMEM((NPW + 16,), jnp.int32),
            pltpu.VMEM((B,), jnp.int32),
            pltpu.VMEM((B + 16,), jnp.int32),
            pltpu.VMEM((B, D), jnp.float32),
            pltpu.VMEM((NPW + 1, D), jnp.float32),
            pltpu.SemaphoreType.DMA,
        ],
    )(rp_pad, col_pad, node_feat)
    return out[:N_NODES]
